# Initial kernel scaffold; baseline (speedup 1.0000x reference)
import jax
import jax.numpy as jnp
from jax.experimental import pallas as pl


def kernel(x, edge_index, edge_weight, segment_ids, W, b, Wd, bd):
    raise NotImplementedError("write your pallas kernel here")


# speedup vs baseline: 2.8864x; 2.8864x over previous
import functools

import jax
import jax.numpy as jnp
from jax import lax
from jax.experimental import pallas as pl
from jax.experimental.pallas import tpu as pltpu
from jax.experimental.pallas import tpu_sc as plsc

N = 10000
E = 320000
F = 128
C = 64
G = 64
L = 4

NC = 2
NS = 16
NW = NC * NS

CHUNK = 128
EPW = 10240
E_PAD = EPW * NW
NCHUNKS = EPW // CHUNK
N_PAD = 10240
ROWS_PER_TILE = N_PAD // NS
ZCOPIES = ROWS_PER_TILE // CHUNK
CP = 128


def _matmul_body(x_ref, w_ref, o_ref):
    o_ref[...] = jnp.dot(x_ref[...], w_ref[...],
                         preferred_element_type=jnp.float32)


def _xw(x, Wp):
    return pl.pallas_call(
        _matmul_body,
        out_shape=jax.ShapeDtypeStruct((N, CP), jnp.float32),
    )(x, Wp)


def _edge_body(xw_hbm, src_hbm, dst_hbm, ew_hbm, out_hbm,
               idx_v, dst_v, w_v, rows_v, agg_sh, sem):
    cid = lax.axis_index("c")
    sid = lax.axis_index("s")
    wid = cid * NS + sid

    def _zero_row(i, _):
        for j in range(C // 16):
            rows_v[i, pl.ds(16 * j, 16)] = jnp.zeros((16,), jnp.float32)
        return 0
    lax.fori_loop(0, CHUNK, _zero_row, 0)
    for z in range(ZCOPIES):
        pltpu.sync_copy(rows_v,
                        agg_sh.at[pl.ds(sid * ROWS_PER_TILE + z * CHUNK, CHUNK)])
    plsc.subcore_barrier()

    def _chunk(c, _):
        pltpu.sync_copy(src_hbm.at[wid, c], idx_v)
        pltpu.sync_copy(dst_hbm.at[wid, c], dst_v)
        pltpu.sync_copy(ew_hbm.at[wid, c], w_v)
        pltpu.async_copy(xw_hbm.at[idx_v], rows_v, sem).wait()

        def _scale(k, _):
            wv = w_v[pl.ds(k * 16, 16)]
            for r in range(16):
                w = wv[r]
                i = k * 16 + r
                for j in range(C // 16):
                    sl = pl.ds(16 * j, 16)
                    rows_v[i, sl] = rows_v[i, sl] * w
            return 0
        lax.fori_loop(0, CHUNK // 16, _scale, 0)

        pltpu.sync_copy(rows_v, agg_sh.at[dst_v], add=True)
        return 0
    lax.fori_loop(0, NCHUNKS, _chunk, 0)

    plsc.subcore_barrier()
    pltpu.sync_copy(agg_sh.at[pl.ds(sid * ROWS_PER_TILE, ROWS_PER_TILE)],
                    out_hbm.at[cid, pl.ds(sid * ROWS_PER_TILE, ROWS_PER_TILE)])


def _edge_aggregate(xw, src, dst, ew):
    mesh = plsc.VectorSubcoreMesh(core_axis_name="c", subcore_axis_name="s")
    run = pl.kernel(
        _edge_body,
        out_type=jax.ShapeDtypeStruct((NC, N_PAD, CP), jnp.float32),
        mesh=mesh,
        scratch_types=[
            pltpu.VMEM((CHUNK,), jnp.int32),
            pltpu.VMEM((CHUNK,), jnp.int32),
            pltpu.VMEM((CHUNK,), jnp.float32),
            pltpu.VMEM((CHUNK, CP), jnp.float32),
            pltpu.VMEM_SHARED((N_PAD, CP), jnp.float32),
            pltpu.SemaphoreType.DMA,
        ],
    )
    return run(xw, src, dst, ew)


def _pool_body(p_ref, seg_ref, b_ref, wd_ref, bd_ref, o_ref):
    agg = p_ref[0, :, :C] + p_ref[1, :, :C]
    h = jax.nn.relu(agg + b_ref[...])
    seg = seg_ref[...]
    gids = lax.broadcasted_iota(jnp.int32, (G, N_PAD), 0)
    oh = (gids == seg).astype(jnp.float32)
    sums = jnp.dot(oh, h, preferred_element_type=jnp.float32)
    counts = jnp.sum(oh, axis=1, keepdims=True)
    pooled = sums / jnp.maximum(counts, 1.0)
    logits = jnp.dot(pooled, wd_ref[...],
                     preferred_element_type=jnp.float32) + bd_ref[...]
    m = jnp.max(logits, axis=-1, keepdims=True)
    e = jnp.exp(logits - m)
    o_ref[...] = e / jnp.sum(e, axis=-1, keepdims=True)


def _pool_dense(partials, seg2d, b, Wd, bd):
    return pl.pallas_call(
        _pool_body,
        out_shape=jax.ShapeDtypeStruct((G, L), jnp.float32),
    )(partials, seg2d, b, Wd, bd)


def kernel(x, edge_index, edge_weight, segment_ids, W, b, Wd, bd):
    src = edge_index[0].astype(jnp.int32)
    dst = edge_index[1].astype(jnp.int32)
    ew = edge_weight.astype(jnp.float32)

    pad = E_PAD - E
    src = jnp.concatenate([src, jnp.zeros((pad,), jnp.int32)]).reshape(NW, NCHUNKS, CHUNK)
    dst = jnp.concatenate([dst, jnp.zeros((pad,), jnp.int32)]).reshape(NW, NCHUNKS, CHUNK)
    ew = jnp.concatenate([ew, jnp.zeros((pad,), jnp.float32)]).reshape(NW, NCHUNKS, CHUNK)

    seg = segment_ids.astype(jnp.int32)
    seg2d = jnp.concatenate([seg, jnp.full((N_PAD - N,), -1, jnp.int32)]).reshape(1, N_PAD)

    Wp = jnp.concatenate([W, jnp.zeros((F, CP - C), jnp.float32)], axis=1)
    xw = _xw(x, Wp)
    partials = _edge_aggregate(xw, src, dst, ew)
    return _pool_dense(partials, seg2d, b, Wd, bd)


# speedup vs baseline: 3.2554x; 1.1279x over previous
import functools

import jax
import jax.numpy as jnp
from jax import lax
from jax.experimental import pallas as pl
from jax.experimental.pallas import tpu as pltpu
from jax.experimental.pallas import tpu_sc as plsc

N = 10000
E = 320000
F = 128
C = 64
G = 64
L = 4

NC = 2
NS = 16
NW = NC * NS

CHUNK = 64
EPW = 10240
E_PAD = EPW * NW
NCHUNKS = EPW // CHUNK
N_PAD = 10240
ROWS_PER_TILE = N_PAD // NS
ZCOPIES = ROWS_PER_TILE // CHUNK
CP = 128


def _matmul_body(x_ref, w_ref, o_ref):
    o_ref[...] = jnp.dot(x_ref[...], w_ref[...],
                         preferred_element_type=jnp.float32)


def _xw(x, Wp):
    return pl.pallas_call(
        _matmul_body,
        out_shape=jax.ShapeDtypeStruct((N, CP), jnp.float32),
    )(x, Wp)


NB = 4
PD = 2


def _edge_body(xw_hbm, src_hbm, dst_hbm, ew_hbm, out_hbm,
               d_ring, src_ring, w_ring, rows, agg_sh, gsem, ssem, isem, dsem):
    cid = lax.axis_index("c")
    sid = lax.axis_index("s")
    wid = cid * NS + sid

    zbuf = rows.at[0]
    def _zero_row(i, _):
        for j in range(CP // 16):
            zbuf[i, pl.ds(16 * j, 16)] = jnp.zeros((16,), jnp.float32)
        return 0
    lax.fori_loop(0, CHUNK, _zero_row, 0)
    for z in range(ZCOPIES):
        pltpu.sync_copy(zbuf,
                        agg_sh.at[pl.ds(sid * ROWS_PER_TILE + z * CHUNK, CHUNK)])
    plsc.subcore_barrier()

    def _iload(c, b):
        pltpu.async_copy(src_hbm.at[wid, c], src_ring.at[b], isem.at[b])
        pltpu.async_copy(ew_hbm.at[wid, c], w_ring.at[b], isem.at[b])

    def _iload_wait(b):
        pltpu.make_async_copy(src_hbm.at[wid, 0], src_ring.at[b],
                              isem.at[b]).wait()
        pltpu.make_async_copy(ew_hbm.at[wid, 0], w_ring.at[b],
                              isem.at[b]).wait()

    def _dload(c, b):
        pltpu.async_copy(dst_hbm.at[wid, c], d_ring.at[b], dsem.at[b])

    def _dload_wait(b):
        pltpu.make_async_copy(dst_hbm.at[wid, 0], d_ring.at[b],
                              dsem.at[b]).wait()

    def _gather(b):
        pltpu.async_copy(xw_hbm.at[src_ring.at[b]], rows.at[b], gsem.at[b])

    def _gather_wait(b):
        pltpu.make_async_copy(xw_hbm.at[src_ring.at[0]], rows.at[b],
                              gsem.at[b]).wait()

    def _scatter(b):
        pltpu.async_copy(rows.at[b], agg_sh.at[d_ring.at[b]], ssem.at[b],
                         add=True)

    def _scatter_wait(b):
        pltpu.make_async_copy(rows.at[b], agg_sh.at[d_ring.at[0]],
                              ssem.at[b]).wait()

    for c in range(PD + 1):
        _iload(c, c)
    for c in range(PD):
        _dload(c, c)
        _iload_wait(c)
        _gather(c)

    def _outer(k, _):
        for b in range(NB):
            c = k * NB + b
            bb = (b + PD) % NB
            bi = (b + PD + 1) % NB

            @pl.when(c + PD + 1 < NCHUNKS)
            def _():
                _iload(c + PD + 1, bi)

            @pl.when(c >= NB - PD)
            def _():
                _scatter_wait(bb)

            @pl.when(c + PD < NCHUNKS)
            def _():
                _dload(c + PD, bb)
                _iload_wait(bb)
                _gather(bb)

            _gather_wait(b)

            def _scale(k16, _):
                wv = w_ring[b, pl.ds(k16 * 16, 16)]
                for r in range(16):
                    w = wv[r]
                    i = k16 * 16 + r
                    for j in range(C // 16):
                        sl = pl.ds(16 * j, 16)
                        rows[b, i, sl] = rows[b, i, sl] * w
                return 0
            lax.fori_loop(0, CHUNK // 16, _scale, 0)

            _dload_wait(b)
            _scatter(b)
        return 0
    lax.fori_loop(0, NCHUNKS // NB, _outer, 0)

    for c in range(NCHUNKS - PD, NCHUNKS):
        _scatter_wait(c % NB)

    plsc.subcore_barrier()
    pltpu.sync_copy(agg_sh.at[pl.ds(sid * ROWS_PER_TILE, ROWS_PER_TILE)],
                    out_hbm.at[cid, pl.ds(sid * ROWS_PER_TILE, ROWS_PER_TILE)])


def _edge_aggregate(xw, src, dst, ew):
    mesh = plsc.VectorSubcoreMesh(core_axis_name="c", subcore_axis_name="s")
    run = pl.kernel(
        _edge_body,
        out_type=jax.ShapeDtypeStruct((NC, N_PAD, CP), jnp.float32),
        mesh=mesh,
        scratch_types=[
            pltpu.VMEM((NB, CHUNK), jnp.int32),
            pltpu.VMEM((NB, CHUNK), jnp.int32),
            pltpu.VMEM((NB, CHUNK), jnp.float32),
            pltpu.VMEM((NB, CHUNK, CP), jnp.float32),
            pltpu.VMEM_SHARED((N_PAD, CP), jnp.float32),
            pltpu.SemaphoreType.DMA((NB,)),
            pltpu.SemaphoreType.DMA((NB,)),
            pltpu.SemaphoreType.DMA((NB,)),
            pltpu.SemaphoreType.DMA((NB,)),
        ],
    )
    return run(xw, src, dst, ew)


def _pool_body(p_ref, seg_ref, b_ref, wd_ref, bd_ref, o_ref):
    agg = p_ref[0, :, :C] + p_ref[1, :, :C]
    h = jax.nn.relu(agg + b_ref[...])
    seg = seg_ref[...]
    gids = lax.broadcasted_iota(jnp.int32, (G, N_PAD), 0)
    oh = (gids == seg).astype(jnp.float32)
    sums = jnp.dot(oh, h, preferred_element_type=jnp.float32)
    counts = jnp.sum(oh, axis=1, keepdims=True)
    pooled = sums / jnp.maximum(counts, 1.0)
    logits = jnp.dot(pooled, wd_ref[...],
                     preferred_element_type=jnp.float32) + bd_ref[...]
    m = jnp.max(logits, axis=-1, keepdims=True)
    e = jnp.exp(logits - m)
    o_ref[...] = e / jnp.sum(e, axis=-1, keepdims=True)


def _pool_dense(partials, seg2d, b, Wd, bd):
    return pl.pallas_call(
        _pool_body,
        out_shape=jax.ShapeDtypeStruct((G, L), jnp.float32),
    )(partials, seg2d, b, Wd, bd)


def kernel(x, edge_index, edge_weight, segment_ids, W, b, Wd, bd):
    src = edge_index[0].astype(jnp.int32)
    dst = edge_index[1].astype(jnp.int32)
    ew = edge_weight.astype(jnp.float32)

    pad = E_PAD - E
    src = jnp.concatenate([src, jnp.zeros((pad,), jnp.int32)]).reshape(NW, NCHUNKS, CHUNK)
    dst = jnp.concatenate([dst, jnp.zeros((pad,), jnp.int32)]).reshape(NW, NCHUNKS, CHUNK)
    ew = jnp.concatenate([ew, jnp.zeros((pad,), jnp.float32)]).reshape(NW, NCHUNKS, CHUNK)

    seg = segment_ids.astype(jnp.int32)
    seg2d = jnp.concatenate([seg, jnp.full((N_PAD - N,), -1, jnp.int32)]).reshape(1, N_PAD)

    Wp = jnp.concatenate([W, jnp.zeros((F, CP - C), jnp.float32)], axis=1)
    xw = _xw(x, Wp)
    partials = _edge_aggregate(xw, src, dst, ew)
    return _pool_dense(partials, seg2d, b, Wd, bd)


# speedup vs baseline: 4.2506x; 1.3057x over previous
import functools

import jax
import jax.numpy as jnp
from jax import lax
from jax.experimental import pallas as pl
from jax.experimental.pallas import tpu as pltpu
from jax.experimental.pallas import tpu_sc as plsc

N = 10000
E = 320000
F = 128
C = 64
G = 64
L = 4

NC = 2
NS = 16
NW = NC * NS

CHUNK = 80
EPW = 10240
E_PAD = EPW * NW
NCHUNKS = EPW // CHUNK
N_PAD = 10240
ROWS_PER_TILE = N_PAD // NS
ZCOPIES = ROWS_PER_TILE // CHUNK
CP = 128


def _matmul_body(x_ref, w_ref, o_ref):
    o_ref[...] = jnp.dot(x_ref[...], w_ref[...],
                         preferred_element_type=jnp.float32)


def _xw(x, Wp):
    return pl.pallas_call(
        _matmul_body,
        out_shape=jax.ShapeDtypeStruct((N, CP), jnp.float32),
    )(x, Wp)


NB = 4
ND = 8
PD = 2
UN = 8


def _edge_body(xw_hbm, src_hbm, dst_hbm, ew_hbm, out_hbm,
               d_ring, src_ring, w_ring, rows, agg_sh,
               gsem, ssem, isem, dsem):
    cid = lax.axis_index("c")
    sid = lax.axis_index("s")
    wid = cid * NS + sid

    zbuf = rows.at[0]
    def _zero_row(i, _):
        for j in range(CP // 16):
            zbuf[i, pl.ds(16 * j, 16)] = jnp.zeros((16,), jnp.float32)
        return 0
    lax.fori_loop(0, CHUNK, _zero_row, 0)
    for z in range(ZCOPIES):
        pltpu.sync_copy(zbuf,
                        agg_sh.at[pl.ds(sid * ROWS_PER_TILE + z * CHUNK, CHUNK)])
    plsc.subcore_barrier()

    def _iload(c, b):
        pltpu.async_copy(src_hbm.at[wid, c], src_ring.at[b], isem.at[b])
        pltpu.async_copy(ew_hbm.at[wid, c], w_ring.at[b], isem.at[b])

    def _iload_wait(b):
        pltpu.make_async_copy(src_hbm.at[wid, 0], src_ring.at[b],
                              isem.at[b]).wait()
        pltpu.make_async_copy(ew_hbm.at[wid, 0], w_ring.at[b],
                              isem.at[b]).wait()

    def _dload(c, b):
        pltpu.async_copy(dst_hbm.at[wid, c], d_ring.at[b], dsem.at[b])

    def _dload_wait(b):
        pltpu.make_async_copy(dst_hbm.at[wid, 0], d_ring.at[b],
                              dsem.at[b]).wait()

    def _gather(b):
        pltpu.async_copy(xw_hbm.at[src_ring.at[b]], rows.at[b], gsem.at[b])

    def _gather_wait(b):
        pltpu.make_async_copy(xw_hbm.at[src_ring.at[0]], rows.at[b],
                              gsem.at[b]).wait()

    def _scatter(b, bd):
        pltpu.async_copy(rows.at[b], agg_sh.at[d_ring.at[bd]], ssem.at[b],
                         add=True)

    def _scatter_wait(b):
        pltpu.make_async_copy(rows.at[b], agg_sh.at[d_ring.at[0]],
                              ssem.at[b]).wait()

    for c in range(PD + 1):
        _iload(c, c)
    for c in range(NB):
        _dload(c, c)
    for c in range(PD):
        _iload_wait(c)
        _gather(c)

    def _outer(k, _):
        for u in range(UN):
            c = k * UN + u
            b = u % NB

            @pl.when(c + PD + 1 < NCHUNKS)
            def _():
                _iload(c + PD + 1, (u + PD + 1) % NB)

            @pl.when(c >= NB - PD)
            def _():
                _scatter_wait((u + PD) % NB)

            @pl.when(c + NB < NCHUNKS)
            def _():
                _dload(c + NB, (u + NB) % ND)

            @pl.when(c + PD < NCHUNKS)
            def _():
                _iload_wait((u + PD) % NB)
                _gather((u + PD) % NB)

            _gather_wait(b)

            def _scale(k16, _):
                wv = w_ring[b, pl.ds(k16 * 16, 16)]
                for r in range(16):
                    w = wv[r]
                    i = k16 * 16 + r
                    for j in range(C // 16):
                        sl = pl.ds(16 * j, 16)
                        rows[b, i, sl] = rows[b, i, sl] * w
                return 0
            lax.fori_loop(0, CHUNK // 16, _scale, 0)

            _dload_wait(u % ND)
            _scatter(b, u % ND)
        return 0
    lax.fori_loop(0, NCHUNKS // UN, _outer, 0)

    for c in range(NCHUNKS - PD, NCHUNKS):
        _scatter_wait(c % NB)

    plsc.subcore_barrier()
    pltpu.sync_copy(agg_sh.at[pl.ds(sid * ROWS_PER_TILE, ROWS_PER_TILE)],
                    out_hbm.at[cid, pl.ds(sid * ROWS_PER_TILE, ROWS_PER_TILE)])


def _edge_aggregate(xw, src, dst, ew):
    mesh = plsc.VectorSubcoreMesh(core_axis_name="c", subcore_axis_name="s")
    run = pl.kernel(
        _edge_body,
        out_type=jax.ShapeDtypeStruct((NC, N_PAD, CP), jnp.float32),
        mesh=mesh,
        scratch_types=[
            pltpu.VMEM((ND, CHUNK), jnp.int32),
            pltpu.VMEM((NB, CHUNK), jnp.int32),
            pltpu.VMEM((NB, CHUNK), jnp.float32),
            pltpu.VMEM((NB, CHUNK, CP), jnp.float32),
            pltpu.VMEM_SHARED((N_PAD, CP), jnp.float32),
            pltpu.SemaphoreType.DMA((NB,)),
            pltpu.SemaphoreType.DMA((NB,)),
            pltpu.SemaphoreType.DMA((NB,)),
            pltpu.SemaphoreType.DMA((ND,)),
        ],
    )
    return run(xw, src, dst, ew)


def _pool_body(p_ref, seg_ref, b_ref, wd_ref, bd_ref, o_ref):
    agg = p_ref[0, :, :C] + p_ref[1, :, :C]
    h = jax.nn.relu(agg + b_ref[...])
    seg = seg_ref[...]
    gids = lax.broadcasted_iota(jnp.int32, (G, N_PAD), 0)
    oh = (gids == seg).astype(jnp.float32)
    sums = jnp.dot(oh, h, preferred_element_type=jnp.float32)
    counts = jnp.sum(oh, axis=1, keepdims=True)
    pooled = sums / jnp.maximum(counts, 1.0)
    logits = jnp.dot(pooled, wd_ref[...],
                     preferred_element_type=jnp.float32) + bd_ref[...]
    m = jnp.max(logits, axis=-1, keepdims=True)
    e = jnp.exp(logits - m)
    o_ref[...] = e / jnp.sum(e, axis=-1, keepdims=True)


def _pool_dense(partials, seg2d, b, Wd, bd):
    return pl.pallas_call(
        _pool_body,
        out_shape=jax.ShapeDtypeStruct((G, L), jnp.float32),
    )(partials, seg2d, b, Wd, bd)


def kernel(x, edge_index, edge_weight, segment_ids, W, b, Wd, bd):
    src = edge_index[0].astype(jnp.int32)
    dst = edge_index[1].astype(jnp.int32)
    ew = edge_weight.astype(jnp.float32)

    pad = E_PAD - E
    src = jnp.concatenate([src, jnp.zeros((pad,), jnp.int32)]).reshape(NW, NCHUNKS, CHUNK)
    dst = jnp.concatenate([dst, jnp.zeros((pad,), jnp.int32)]).reshape(NW, NCHUNKS, CHUNK)
    ew = jnp.concatenate([ew, jnp.zeros((pad,), jnp.float32)]).reshape(NW, NCHUNKS, CHUNK)

    seg = segment_ids.astype(jnp.int32)
    seg2d = jnp.concatenate([seg, jnp.full((N_PAD - N,), -1, jnp.int32)]).reshape(1, N_PAD)

    Wp = jnp.concatenate([W, jnp.zeros((F, CP - C), jnp.float32)], axis=1)
    xw = _xw(x, Wp)
    partials = _edge_aggregate(xw, src, dst, ew)
    return _pool_dense(partials, seg2d, b, Wd, bd)
